# baseline (speedup 1.0000x reference)
import functools

import jax
import jax.numpy as jnp
from jax import lax
from jax.experimental import pallas as pl
from jax.experimental.pallas import tpu as pltpu
from jax.experimental.pallas import tpu_sc as plsc

EMBED_DIM = 64
BATCH = 4096
SEQ_LEN = 50
TOT = BATCH * SEQ_LEN
NC = 2
NS = 16
NW = NC * NS
PER_W = TOT // NW
CHUNK = 128
NCHUNK = PER_W // CHUNK

_mesh = plsc.VectorSubcoreMesh(core_axis_name="c", subcore_axis_name="s")


@functools.partial(
    pl.kernel,
    mesh=_mesh,
    out_type=jax.ShapeDtypeStruct((TOT, EMBED_DIM), jnp.float32),
    scratch_types=[
        pltpu.VMEM((NCHUNK, CHUNK), jnp.int32),
        pltpu.VMEM((CHUNK, EMBED_DIM), jnp.float32),
        pltpu.SemaphoreType.DMA,
    ],
    compiler_params=pltpu.CompilerParams(use_tc_tiling_on_sc=False),
)
def _sc_gather(ids_hbm, table_hbm, out_hbm, idx_v, rows_v, sem):
    wid = lax.axis_index("s") * NC + lax.axis_index("c")
    base = wid * PER_W
    pltpu.sync_copy(ids_hbm.at[wid], idx_v)

    def body(j, carry):
        pltpu.async_copy(table_hbm.at[idx_v.at[j]], rows_v, sem).wait()
        pltpu.sync_copy(rows_v, out_hbm.at[pl.ds(base + j * CHUNK, CHUNK)])
        return carry

    lax.fori_loop(0, NCHUNK, body, 0)


def kernel(input_ids, table):
    ids = input_ids.reshape(NW, NCHUNK, CHUNK).astype(jnp.int32)
    out = _sc_gather(ids, table)
    return out.reshape(BATCH, SEQ_LEN, EMBED_DIM)


# speedup vs baseline: 1.0379x; 1.0379x over previous
import functools

import jax
import jax.numpy as jnp
from jax import lax
from jax.experimental import pallas as pl
from jax.experimental.pallas import tpu as pltpu
from jax.experimental.pallas import tpu_sc as plsc

EMBED_DIM = 64
BATCH = 4096
SEQ_LEN = 50
TOT = BATCH * SEQ_LEN
NC = 2
NS = 16
NW = NC * NS
PER_W = TOT // NW
CHUNK = 640
NCHUNK = PER_W // CHUNK

_mesh = plsc.VectorSubcoreMesh(core_axis_name="c", subcore_axis_name="s")


@functools.partial(
    pl.kernel,
    mesh=_mesh,
    out_type=jax.ShapeDtypeStruct((TOT, EMBED_DIM), jnp.float32),
    scratch_types=[
        pltpu.VMEM((NCHUNK, CHUNK), jnp.int32),
        pltpu.VMEM((CHUNK, EMBED_DIM), jnp.float32),
        pltpu.SemaphoreType.DMA,
    ],
    compiler_params=pltpu.CompilerParams(use_tc_tiling_on_sc=False),
)
def _sc_gather(ids_hbm, table_hbm, out_hbm, idx_v, rows_v, sem):
    wid = lax.axis_index("s") * NC + lax.axis_index("c")
    base = wid * PER_W
    pltpu.sync_copy(ids_hbm.at[wid], idx_v)

    def body(j, carry):
        pltpu.async_copy(table_hbm.at[idx_v.at[j]], rows_v, sem).wait()
        pltpu.sync_copy(rows_v, out_hbm.at[pl.ds(base + j * CHUNK, CHUNK)])
        return carry

    lax.fori_loop(0, NCHUNK, body, 0)


def kernel(input_ids, table):
    ids = input_ids.reshape(NW, NCHUNK, CHUNK).astype(jnp.int32)
    out = _sc_gather(ids, table)
    return out.reshape(BATCH, SEQ_LEN, EMBED_DIM)


# speedup vs baseline: 1.0412x; 1.0032x over previous
import functools

import jax
import jax.numpy as jnp
from jax import lax
from jax.experimental import pallas as pl
from jax.experimental.pallas import tpu as pltpu
from jax.experimental.pallas import tpu_sc as plsc

EMBED_DIM = 64
BATCH = 4096
SEQ_LEN = 50
NC = 2
NS = 16
NW = NC * NS
ROWS_W = BATCH // NW
GB = 16
NCH = ROWS_W // GB

_mesh = plsc.VectorSubcoreMesh(core_axis_name="c", subcore_axis_name="s")


@functools.partial(
    pl.kernel,
    mesh=_mesh,
    out_type=jax.ShapeDtypeStruct((BATCH, SEQ_LEN, EMBED_DIM), jnp.float32),
    scratch_types=[
        pltpu.VMEM((ROWS_W, SEQ_LEN), jnp.int32),
        pltpu.VMEM((GB, SEQ_LEN, EMBED_DIM), jnp.float32),
        pltpu.VMEM((GB, SEQ_LEN, EMBED_DIM), jnp.float32),
        pltpu.SemaphoreType.DMA,
        pltpu.SemaphoreType.DMA,
        pltpu.SemaphoreType.DMA,
        pltpu.SemaphoreType.DMA,
    ],
    compiler_params=pltpu.CompilerParams(use_tc_tiling_on_sc=False),
)
def _sc_gather(ids_hbm, table_hbm, out_hbm, idx_v, buf0, buf1, gs0, gs1, os0, os1):
    wid = lax.axis_index("s") * NC + lax.axis_index("c")
    rbase = wid * ROWS_W
    pltpu.sync_copy(ids_hbm.at[pl.ds(rbase, ROWS_W)], idx_v)

    bufs = (buf0, buf1)
    gsems = (gs0, gs1)
    osems = (os0, os1)
    gathers = [None] * NCH
    outs = [None] * NCH

    def start_chunk(j, b):
        return [
            pltpu.async_copy(
                table_hbm.at[idx_v.at[j * GB + i]], bufs[b].at[i], gsems[b])
            for i in range(GB)
        ]

    gathers[0] = start_chunk(0, 0)
    for j in range(NCH):
        b = j & 1
        nb = (j + 1) & 1
        if j + 1 < NCH:
            if j >= 1:
                outs[j - 1].wait()
            gathers[j + 1] = start_chunk(j + 1, nb)
        for h in gathers[j]:
            h.wait()
        outs[j] = pltpu.async_copy(
            bufs[b], out_hbm.at[pl.ds(rbase + j * GB, GB)], osems[b])
    outs[NCH - 2].wait()
    outs[NCH - 1].wait()


def kernel(input_ids, table):
    return _sc_gather(input_ids.astype(jnp.int32), table)
